# baseline (speedup 1.0000x reference)
import functools

import jax
import jax.numpy as jnp
import numpy as np
from jax import lax
from jax.experimental import pallas as pl
from jax.experimental.pallas import tpu as pltpu

B, CIN, COUT, N = 4, 128, 64, 4096
K = N // 4
R, L = 32, 128


def _scores_body(l1_ref, wq_ref, wk_ref, u_ref):
    wq = wq_ref[...]
    wk = wk_ref[...]
    for b in range(B):
        l1b = l1_ref[b]
        q0 = jnp.dot(wq, l1b[:, 0:1])
        k1b = jnp.dot(wk, l1b)
        e = jnp.dot(q0.T, k1b)
        x = e * np.float32(0.125)
        m = jnp.max(x)
        u_ref[b:b + 1, :] = jnp.exp(x - m)


def _scores(l1, Wq, Wk):
    return pl.pallas_call(
        _scores_body,
        out_shape=jax.ShapeDtypeStruct((B, N), jnp.float32),
    )(l1, Wq, Wk)


def _sort_gather_body(u_ref, s_ref, shift_ref, l1_ref, xyz_ref,
                      l1out_ref, p1out_ref):
    att = u_ref[...] / s_ref[...]
    key = ~lax.bitcast_convert_type(att, jnp.uint32)
    row = lax.broadcasted_iota(jnp.int32, (B, R, L), 1)
    lane = lax.broadcasted_iota(jnp.int32, (B, R, L), 2)
    i_full = row * L + lane

    def partner(x, j):
        if j < L:
            lo = (lane & j) == 0
            return jnp.where(lo, jnp.roll(x, -j, axis=2), jnp.roll(x, j, axis=2))
        m = j // L
        xr = x.reshape(B, R // (2 * m), 2, m, L)
        xr = jnp.concatenate([xr[:, :, 1:2], xr[:, :, 0:1]], axis=2)
        return xr.reshape(B, R, L)

    idx = i_full
    k = 2
    while k <= N:
        j = k // 2
        while j >= 1:
            pk = partner(key, j)
            pi = partner(idx, j)
            is_lo = (i_full & j) == 0
            if k < N:
                want_min = ((i_full & k) == 0) == is_lo
            else:
                want_min = is_lo
            gt = (key > pk) | ((key == pk) & (idx > pi))
            take = gt == want_min
            key = jnp.where(take, pk, key)
            idx = jnp.where(take, pi, idx)
            j //= 2
        k *= 2

    shift = shift_ref[0]
    iota_col = lax.broadcasted_iota(jnp.int32, (N, 1), 0)
    dn = (((1,), (0,)), ((), ()))
    for b in range(B):
        l1b = l1_ref[b]
        xyzb = xyz_ref[b]
        for rb in range(K // L):
            sel = idx[b, rb:rb + 1, :].reshape(1, L) + shift
            oh_t = (iota_col == sel).astype(jnp.float32)
            gl = lax.dot_general(l1b, oh_t, dn)
            gp = lax.dot_general(xyzb, oh_t, dn)
            l1out_ref[b, pl.ds(rb * L, L), :] = gl.T
            p1out_ref[b, pl.ds(rb * L, L), :] = gp.T


def _sort_gather(u, s, shift, l1, xyz1):
    return pl.pallas_call(
        _sort_gather_body,
        out_shape=(jax.ShapeDtypeStruct((B, K, CIN), jnp.float32),
                   jax.ShapeDtypeStruct((B, K, 3), jnp.float32)),
        in_specs=[
            pl.BlockSpec(),
            pl.BlockSpec(),
            pl.BlockSpec(memory_space=pltpu.SMEM),
            pl.BlockSpec(),
            pl.BlockSpec(),
        ],
    )(u.reshape(B, R, L), s.reshape(B, 1, 1), shift, l1, xyz1)


def kernel(l1, xyz1, top_k, Wq, Wk):
    u = _scores(l1, Wq, Wk)
    s = jnp.sum(u, axis=-1, keepdims=True)
    shift = (jnp.asarray(top_k, dtype=jnp.int32) - K).reshape(1)
    return _sort_gather(u, s, shift, l1, xyz1)
